# baseline (speedup 1.0000x reference)
import functools

import jax
import jax.numpy as jnp
from jax import lax
from jax.experimental import pallas as pl
from jax.experimental.pallas import tpu as pltpu
from jax.experimental.pallas import tpu_sc as plsc

B, P, N, D = 8, 64, 1024, 64
ROW_LEN = P * D
ROWS = B * P * N * D // ROW_LEN
BLOCK_ROWS = 512


def _sc_gather_kernel(idx_hbm, table_hbm, out_hbm, idx_v, rows_v, sem):
    c = lax.axis_index("c")
    s = lax.axis_index("s")

    @pl.when(jnp.logical_and(c == 0, s == 0))
    def _():
        pltpu.sync_copy(idx_hbm, idx_v)
        pltpu.async_copy(table_hbm.at[idx_v], rows_v, sem).wait()
        pltpu.sync_copy(rows_v, out_hbm)


def _sc_gather(index, position_embedding):
    mesh = plsc.VectorSubcoreMesh(core_axis_name="c", subcore_axis_name="s")
    kern = functools.partial(
        pl.kernel,
        mesh=mesh,
        out_type=jax.ShapeDtypeStruct((P, D), jnp.float32),
        scratch_types=[
            pltpu.VMEM((P,), jnp.int32),
            pltpu.VMEM((P, D), jnp.float32),
            pltpu.SemaphoreType.DMA,
        ],
        compiler_params=pltpu.CompilerParams(use_tc_tiling_on_sc=False),
    )(_sc_gather_kernel)
    return kern(index, position_embedding)


def _tc_add_kernel(x_ref, pe_ref, o_ref):
    o_ref[...] = x_ref[...] + pe_ref[...]


def _tc_add(x2, pe_row):
    grid = (ROWS // BLOCK_ROWS,)
    return pl.pallas_call(
        _tc_add_kernel,
        grid=grid,
        in_specs=[
            pl.BlockSpec((BLOCK_ROWS, ROW_LEN), lambda i: (i, 0)),
            pl.BlockSpec((1, ROW_LEN), lambda i: (0, 0)),
        ],
        out_specs=pl.BlockSpec((BLOCK_ROWS, ROW_LEN), lambda i: (i, 0)),
        out_shape=jax.ShapeDtypeStruct((ROWS, ROW_LEN), jnp.float32),
    )(x2, pe_row)


@jax.jit
def kernel(input_data, index, position_embedding):
    idx = index.astype(jnp.int32)
    tile = _sc_gather(idx, position_embedding)
    x2 = input_data.reshape(ROWS, ROW_LEN)
    out2 = _tc_add(x2, tile.reshape(1, ROW_LEN))
    return out2.reshape(B, N, P, D)


# speedup vs baseline: 1.0012x; 1.0012x over previous
import functools

import jax
import jax.numpy as jnp
from jax import lax
from jax.experimental import pallas as pl
from jax.experimental.pallas import tpu as pltpu
from jax.experimental.pallas import tpu_sc as plsc

B, P, N, D = 8, 64, 1024, 64
ROW_LEN = P * D
ROWS = B * P * N * D // ROW_LEN
BLOCK_ROWS = 512


def _sc_gather_kernel(idx_hbm, table_hbm, out_hbm, idx_v, rows_v, sem):
    c = lax.axis_index("c")
    s = lax.axis_index("s")

    @pl.when(jnp.logical_and(c == 0, s == 0))
    def _():
        pltpu.sync_copy(idx_hbm, idx_v)
        pltpu.async_copy(table_hbm.at[idx_v], rows_v, sem).wait()
        pltpu.sync_copy(rows_v, out_hbm)


def _sc_gather(index, table128):
    mesh = plsc.VectorSubcoreMesh(core_axis_name="c", subcore_axis_name="s")
    kern = functools.partial(
        pl.kernel,
        mesh=mesh,
        out_type=jax.ShapeDtypeStruct((P, 128), jnp.float32),
        scratch_types=[
            pltpu.VMEM((P,), jnp.int32),
            pltpu.VMEM((P, 128), jnp.float32),
            pltpu.SemaphoreType.DMA,
        ],
    )(_sc_gather_kernel)
    return kern(index, table128)


def _tc_add_kernel(x_ref, pe_ref, o_ref):
    o_ref[...] = x_ref[...] + pe_ref[...]


def _tc_add(x2, pe_row):
    grid = (ROWS // BLOCK_ROWS,)
    return pl.pallas_call(
        _tc_add_kernel,
        grid=grid,
        in_specs=[
            pl.BlockSpec((BLOCK_ROWS, ROW_LEN), lambda i: (i, 0)),
            pl.BlockSpec((1, ROW_LEN), lambda i: (0, 0)),
        ],
        out_specs=pl.BlockSpec((BLOCK_ROWS, ROW_LEN), lambda i: (i, 0)),
        out_shape=jax.ShapeDtypeStruct((ROWS, ROW_LEN), jnp.float32),
    )(x2, pe_row)


@jax.jit
def kernel(input_data, index, position_embedding):
    idx = index.astype(jnp.int32)
    table128 = jnp.pad(position_embedding, ((0, 0), (0, 128 - D)))
    tile128 = _sc_gather(idx, table128)
    pe_row = tile128[:, :D].reshape(1, ROW_LEN)
    x2 = input_data.reshape(ROWS, ROW_LEN)
    out2 = _tc_add(x2, pe_row)
    return out2.reshape(B, N, P, D)


# speedup vs baseline: 1.0018x; 1.0006x over previous
import jax
import jax.numpy as jnp
from jax.experimental import pallas as pl
from jax.experimental.pallas import tpu as pltpu

B, P, N, D = 8, 64, 1024, 64
ROW_LEN = P * D
ROWS = B * P * N * D // ROW_LEN
BLOCK_ROWS = 512
MAX_LEN = 1000


def _add_kernel(idx_ref, x_ref, table_ref, o_ref, pe_scr):
    @pl.when(pl.program_id(0) == 0)
    def _():
        rows = [table_ref[pl.ds(idx_ref[i], 1), :] for i in range(P)]
        pe_scr[...] = jnp.concatenate(rows, axis=1)

    o_ref[...] = x_ref[...] + pe_scr[...]


def _tc_fused(x2, idx, table):
    grid = (ROWS // BLOCK_ROWS,)
    return pl.pallas_call(
        _add_kernel,
        grid=grid,
        in_specs=[
            pl.BlockSpec(memory_space=pltpu.SMEM),
            pl.BlockSpec((BLOCK_ROWS, ROW_LEN), lambda i: (i, 0)),
            pl.BlockSpec((MAX_LEN, D), lambda i: (0, 0)),
        ],
        out_specs=pl.BlockSpec((BLOCK_ROWS, ROW_LEN), lambda i: (i, 0)),
        out_shape=jax.ShapeDtypeStruct((ROWS, ROW_LEN), jnp.float32),
        scratch_shapes=[pltpu.VMEM((1, ROW_LEN), jnp.float32)],
    )(idx, x2, table)


@jax.jit
def kernel(input_data, index, position_embedding):
    idx = index.astype(jnp.int32)
    x2 = input_data.reshape(ROWS, ROW_LEN)
    out2 = _tc_fused(x2, idx, position_embedding)
    return out2.reshape(B, N, P, D)
